# Initial kernel scaffold; baseline (speedup 1.0000x reference)
import jax
import jax.numpy as jnp
from jax.experimental import pallas as pl


def kernel(x, W_enc, b_enc, ln_g, ln_b, Wq, Wk, Wv, bq, bk, bv, Wo, bo):
    raise NotImplementedError("write your pallas kernel here")


# speedup vs baseline: 3.3136x; 3.3136x over previous
import jax
import jax.numpy as jnp
import numpy as np
from jax.experimental import pallas as pl

B, T, R, Cg, H, NH = 16, 512, 10, 8, 128, 4
DH = H // NH
BT = B * T
TB = 256


def _body(x_ref, W_enc_ref, b_enc_ref, ln_g_ref, ln_b_ref,
          Wq_ref, Wk_ref, Wv_ref, bq_ref, bk_ref, bv_ref, Wo_ref, bo_ref,
          gf_ref, rf_ref):
    x = x_ref[...]
    inv_sqrt2 = np.float32(1.0 / np.sqrt(2.0))
    scale = np.float32(1.0 / np.sqrt(DH))

    nodes_t = []
    for r in range(R):
        xr = x[:, r * Cg:(r + 1) * Cg]
        h = jax.lax.dot_general(xr, W_enc_ref[r],
                                (((1,), (0,)), ((), ())),
                                preferred_element_type=jnp.float32)
        h = h + b_enc_ref[r:r + 1, :]
        mu = jnp.mean(h, axis=-1, keepdims=True)
        var = jnp.mean((h - mu) * (h - mu), axis=-1, keepdims=True)
        h = (h - mu) * jax.lax.rsqrt(var + 1e-5)
        h = h * ln_g_ref[r:r + 1, :] + ln_b_ref[r:r + 1, :]
        g = 0.5 * h * (1.0 + jax.lax.erf(h * inv_sqrt2))
        rf_ref[:, r, :] = g
        nodes_t.append(g.T)

    Wq = Wq_ref[...]
    Wk = Wk_ref[...]
    Wv = Wv_ref[...]
    Wo = Wo_ref[...]
    bq = bq_ref[...]
    bk = bk_ref[...]
    bv = bv_ref[...]
    bo = bo_ref[...]

    def mm(a, b):
        return jax.lax.dot_general(a, b, (((1,), (0,)), ((), ())),
                                   preferred_element_type=jnp.float32)

    qs = [mm(Wq, n) + bq for n in nodes_t]
    ks = [mm(Wk, n) + bk for n in nodes_t]
    vs = [mm(Wv, n) + bv for n in nodes_t]

    def head_sum(p):
        seg = jnp.sum(p.reshape(NH, DH, TB), axis=1, keepdims=True)
        return jnp.broadcast_to(seg, (NH, DH, TB)).reshape(H, TB)

    for r in range(R):
        ls = [head_sum(qs[r] * ks[s]) * scale for s in range(R)]
        m = ls[0]
        for s in range(1, R):
            m = jnp.maximum(m, ls[s])
        es = [jnp.exp(l - m) for l in ls]
        z = es[0]
        for s in range(1, R):
            z = z + es[s]
        o = es[0] * vs[0]
        for s in range(1, R):
            o = o + es[s] * vs[s]
        o = o / z
        out_t = mm(Wo, o) + bo + nodes_t[r]
        gf_ref[:, r * H:(r + 1) * H] = out_t.T


def kernel(x, W_enc, b_enc, ln_g, ln_b, Wq, Wk, Wv, bq, bk, bv, Wo, bo):
    x2 = x.reshape(BT, R * Cg)
    grid = (BT // TB,)
    full = lambda i: (0, 0)
    gf2, rf2 = pl.pallas_call(
        _body,
        grid=grid,
        in_specs=[
            pl.BlockSpec((TB, R * Cg), lambda i: (i, 0)),
            pl.BlockSpec((R, Cg, H), lambda i: (0, 0, 0)),
            pl.BlockSpec((R, H), full),
            pl.BlockSpec((R, H), full),
            pl.BlockSpec((R, H), full),
            pl.BlockSpec((H, H), full),
            pl.BlockSpec((H, H), full),
            pl.BlockSpec((H, H), full),
            pl.BlockSpec((H, 1), full),
            pl.BlockSpec((H, 1), full),
            pl.BlockSpec((H, 1), full),
            pl.BlockSpec((H, H), full),
            pl.BlockSpec((H, 1), full),
        ],
        out_specs=[
            pl.BlockSpec((TB, R * H), lambda i: (i, 0)),
            pl.BlockSpec((TB, R, H), lambda i: (i, 0, 0)),
        ],
        out_shape=[
            jax.ShapeDtypeStruct((BT, R * H), jnp.float32),
            jax.ShapeDtypeStruct((BT, R, H), jnp.float32),
        ],
    )(x2, W_enc, b_enc, ln_g, ln_b, Wq, Wk, Wv,
      bq.reshape(H, 1), bk.reshape(H, 1), bv.reshape(H, 1),
      Wo, bo.reshape(H, 1))
    return gf2.reshape(B, T, R * H), rf2.reshape(B, T, R, H)


# speedup vs baseline: 3.4668x; 1.0462x over previous
import jax
import jax.numpy as jnp
import numpy as np
from jax.experimental import pallas as pl

B, T, R, Cg, H, NH = 16, 512, 10, 8, 128, 4
DH = H // NH
BT = B * T
TB = 256


def _body(x_ref, W_enc_ref, b_enc_ref, ln_g_ref, ln_b_ref,
          Wq_ref, Wk_ref, Wv_ref, bq_ref, bk_ref, bv_ref, Wo_ref, bo_ref,
          gf_ref, rf_ref):
    x = x_ref[...]
    inv_sqrt2 = np.float32(1.0 / np.sqrt(2.0))
    scale = np.float32(1.0 / np.sqrt(DH))
    ones_h = jnp.full((H, H), np.float32(1.0 / H), dtype=jnp.float32)

    def mean_lanes(a):
        return jax.lax.dot_general(a, ones_h, (((1,), (0,)), ((), ())),
                                   preferred_element_type=jnp.float32)

    nodes_t = []
    for r in range(R):
        xr = x[:, r * Cg:(r + 1) * Cg]
        h = jax.lax.dot_general(xr, W_enc_ref[r],
                                (((1,), (0,)), ((), ())),
                                preferred_element_type=jnp.float32)
        h = h + b_enc_ref[r:r + 1, :]
        mu = mean_lanes(h)
        d = h - mu
        var = mean_lanes(d * d)
        h = d * jax.lax.rsqrt(var + 1e-5)
        h = h * ln_g_ref[r:r + 1, :] + ln_b_ref[r:r + 1, :]
        g = 0.5 * h * (1.0 + jax.lax.erf(h * inv_sqrt2))
        rf_ref[:, r, :] = g
        nodes_t.append(g.T)

    Wq = Wq_ref[...]
    Wk = Wk_ref[...]
    Wv = Wv_ref[...]
    Wo = Wo_ref[...]
    bq = bq_ref[...]
    bk = bk_ref[...]
    bv = bv_ref[...]
    bo = bo_ref[...]

    def mm(a, b):
        return jax.lax.dot_general(a, b, (((1,), (0,)), ((), ())),
                                   preferred_element_type=jnp.float32)

    qs = [mm(Wq, n) + bq for n in nodes_t]
    ks = [mm(Wk, n) + bk for n in nodes_t]
    vs = [mm(Wv, n) + bv for n in nodes_t]

    for r in range(R):
        segs = [jnp.sum((qs[r] * ks[s]).reshape(NH, DH, TB), axis=1)
                for s in range(R)]
        l = jnp.stack(segs, axis=1) * scale
        m = jnp.max(l, axis=1, keepdims=True)
        e = jnp.exp(l - m)
        z = jnp.sum(e, axis=1, keepdims=True)
        w = e / z
        o = None
        for s in range(R):
            wb = jnp.broadcast_to(w[:, s:s + 1, :], (NH, DH, TB)).reshape(H, TB)
            o = wb * vs[s] if o is None else o + wb * vs[s]
        out_t = mm(Wo, o) + bo + nodes_t[r]
        gf_ref[:, r * H:(r + 1) * H] = out_t.T


def kernel(x, W_enc, b_enc, ln_g, ln_b, Wq, Wk, Wv, bq, bk, bv, Wo, bo):
    x2 = x.reshape(BT, R * Cg)
    grid = (BT // TB,)
    full = lambda i: (0, 0)
    gf2, rf2 = pl.pallas_call(
        _body,
        grid=grid,
        in_specs=[
            pl.BlockSpec((TB, R * Cg), lambda i: (i, 0)),
            pl.BlockSpec((R, Cg, H), lambda i: (0, 0, 0)),
            pl.BlockSpec((R, H), full),
            pl.BlockSpec((R, H), full),
            pl.BlockSpec((R, H), full),
            pl.BlockSpec((H, H), full),
            pl.BlockSpec((H, H), full),
            pl.BlockSpec((H, H), full),
            pl.BlockSpec((H, 1), full),
            pl.BlockSpec((H, 1), full),
            pl.BlockSpec((H, 1), full),
            pl.BlockSpec((H, H), full),
            pl.BlockSpec((H, 1), full),
        ],
        out_specs=[
            pl.BlockSpec((TB, R * H), lambda i: (i, 0)),
            pl.BlockSpec((TB, R, H), lambda i: (i, 0, 0)),
        ],
        out_shape=[
            jax.ShapeDtypeStruct((BT, R * H), jnp.float32),
            jax.ShapeDtypeStruct((BT, R, H), jnp.float32),
        ],
    )(x2, W_enc, b_enc, ln_g, ln_b, Wq, Wk, Wv,
      bq.reshape(H, 1), bk.reshape(H, 1), bv.reshape(H, 1),
      Wo, bo.reshape(H, 1))
    return gf2.reshape(B, T, R * H), rf2.reshape(B, T, R, H)


# speedup vs baseline: 3.6983x; 1.0668x over previous
import jax
import jax.numpy as jnp
import numpy as np
from jax.experimental import pallas as pl

B, T, R, Cg, H, NH = 16, 512, 10, 8, 128, 4
DH = H // NH
BT = B * T
TB = 256


def _body(x_ref, W_enc_ref, b_enc_ref, ln_g_ref, ln_b_ref,
          Wq_ref, Wk_ref, Wv_ref, bq_ref, bk_ref, bv_ref, Wo_ref, bo_ref,
          gf_ref, rf_ref):
    x = x_ref[0]
    inv_sqrt2 = np.float32(1.0 / np.sqrt(2.0))
    scale = np.float32(1.0 / np.sqrt(DH))
    ones_h = jnp.full((H, H), np.float32(1.0 / H), dtype=jnp.float32)

    def mean_lanes(a):
        return jax.lax.dot_general(a, ones_h, (((1,), (0,)), ((), ())),
                                   preferred_element_type=jnp.float32)

    nodes_t = []
    for r in range(R):
        xr = x[:, r * Cg:(r + 1) * Cg]
        h = jax.lax.dot_general(xr, W_enc_ref[r],
                                (((1,), (0,)), ((), ())),
                                preferred_element_type=jnp.float32)
        h = h + b_enc_ref[r:r + 1, :]
        mu = mean_lanes(h)
        d = h - mu
        var = mean_lanes(d * d)
        h = d * jax.lax.rsqrt(var + 1e-5)
        h = h * ln_g_ref[r:r + 1, :] + ln_b_ref[r:r + 1, :]
        g = 0.5 * h * (1.0 + jax.lax.erf(h * inv_sqrt2))
        rf_ref[0, :, r, :] = g
        nodes_t.append(g.T)

    Wq = Wq_ref[...]
    Wk = Wk_ref[...]
    Wv = Wv_ref[...]
    Wo = Wo_ref[...]
    bq = bq_ref[...]
    bk = bk_ref[...]
    bv = bv_ref[...]
    bo = bo_ref[...]

    def mm(a, b):
        return jax.lax.dot_general(a, b, (((1,), (0,)), ((), ())),
                                   preferred_element_type=jnp.float32)

    qs = [mm(Wq, n) + bq for n in nodes_t]
    ks = [mm(Wk, n) + bk for n in nodes_t]
    vs = [mm(Wv, n) + bv for n in nodes_t]

    for r in range(R):
        segs = [jnp.sum((qs[r] * ks[s]).reshape(NH, DH, TB), axis=1)
                for s in range(R)]
        l = jnp.stack(segs, axis=1) * scale
        m = jnp.max(l, axis=1, keepdims=True)
        e = jnp.exp(l - m)
        z = jnp.sum(e, axis=1, keepdims=True)
        w = e / z
        o = None
        for s in range(R):
            wb = jnp.broadcast_to(w[:, s:s + 1, :], (NH, DH, TB)).reshape(H, TB)
            o = wb * vs[s] if o is None else o + wb * vs[s]
        out_t = mm(Wo, o) + bo + nodes_t[r]
        gf_ref[0, :, r * H:(r + 1) * H] = out_t.T


def kernel(x, W_enc, b_enc, ln_g, ln_b, Wq, Wk, Wv, bq, bk, bv, Wo, bo):
    grid = (B, T // TB)
    full = lambda b, t: (0, 0)
    gf, rf = pl.pallas_call(
        _body,
        grid=grid,
        in_specs=[
            pl.BlockSpec((1, TB, R * Cg), lambda b, t: (b, t, 0)),
            pl.BlockSpec((R, Cg, H), lambda b, t: (0, 0, 0)),
            pl.BlockSpec((R, H), full),
            pl.BlockSpec((R, H), full),
            pl.BlockSpec((R, H), full),
            pl.BlockSpec((H, H), full),
            pl.BlockSpec((H, H), full),
            pl.BlockSpec((H, H), full),
            pl.BlockSpec((H, 1), full),
            pl.BlockSpec((H, 1), full),
            pl.BlockSpec((H, 1), full),
            pl.BlockSpec((H, H), full),
            pl.BlockSpec((H, 1), full),
        ],
        out_specs=[
            pl.BlockSpec((1, TB, R * H), lambda b, t: (b, t, 0)),
            pl.BlockSpec((1, TB, R, H), lambda b, t: (b, t, 0, 0)),
        ],
        out_shape=[
            jax.ShapeDtypeStruct((B, T, R * H), jnp.float32),
            jax.ShapeDtypeStruct((B, T, R, H), jnp.float32),
        ],
    )(x, W_enc, b_enc, ln_g, ln_b, Wq, Wk, Wv,
      bq.reshape(H, 1), bk.reshape(H, 1), bv.reshape(H, 1),
      Wo, bo.reshape(H, 1))
    return gf, rf


# speedup vs baseline: 4.1998x; 1.1356x over previous
import jax
import jax.numpy as jnp
import numpy as np
from jax.experimental import pallas as pl

B, T, R, Cg, H, NH = 16, 512, 10, 8, 128, 4
DH = H // NH
BT = B * T
TB = 512


def _body(x_ref, W_enc_ref, b_enc_ref, ln_g_ref, ln_b_ref,
          Wq_ref, Wk_ref, Wv_ref, bq_ref, bk_ref, bv_ref, Wo_ref, bo_ref,
          gf_ref, rf_ref):
    x = x_ref[0]
    inv_sqrt2 = np.float32(1.0 / np.sqrt(2.0))
    scale = np.float32(1.0 / np.sqrt(DH))
    ones_h = jnp.full((H, H), np.float32(1.0 / H), dtype=jnp.float32)

    def mean_lanes(a):
        return jax.lax.dot_general(a, ones_h, (((1,), (0,)), ((), ())),
                                   preferred_element_type=jnp.float32)

    nodes_t = []
    for r in range(R):
        xr = x[:, r * Cg:(r + 1) * Cg]
        h = jax.lax.dot_general(xr, W_enc_ref[r],
                                (((1,), (0,)), ((), ())),
                                preferred_element_type=jnp.float32)
        h = h + b_enc_ref[r:r + 1, :]
        mu = mean_lanes(h)
        d = h - mu
        var = mean_lanes(d * d)
        h = d * jax.lax.rsqrt(var + 1e-5)
        h = h * ln_g_ref[r:r + 1, :] + ln_b_ref[r:r + 1, :]
        g = 0.5 * h * (1.0 + jax.lax.erf(h * inv_sqrt2))
        rf_ref[0, :, r, :] = g
        nodes_t.append(g.T)

    Wq = Wq_ref[...]
    Wk = Wk_ref[...]
    Wv = Wv_ref[...]
    Wo = Wo_ref[...]
    bq = bq_ref[...]
    bk = bk_ref[...]
    bv = bv_ref[...]
    bo = bo_ref[...]

    def mm(a, b):
        return jax.lax.dot_general(a, b, (((1,), (0,)), ((), ())),
                                   preferred_element_type=jnp.float32)

    qs = [mm(Wq, n) + bq for n in nodes_t]
    ks = [mm(Wk, n) + bk for n in nodes_t]
    vs = [mm(Wv, n) + bv for n in nodes_t]

    for r in range(R):
        segs = [jnp.sum((qs[r] * ks[s]).reshape(NH, DH, TB), axis=1)
                for s in range(R)]
        l = jnp.stack(segs, axis=1) * scale
        m = jnp.max(l, axis=1, keepdims=True)
        e = jnp.exp(l - m)
        z = jnp.sum(e, axis=1, keepdims=True)
        w = e / z
        o = None
        for s in range(R):
            wb = jnp.broadcast_to(w[:, s:s + 1, :], (NH, DH, TB)).reshape(H, TB)
            o = wb * vs[s] if o is None else o + wb * vs[s]
        out_t = mm(Wo, o) + bo + nodes_t[r]
        gf_ref[0, :, r * H:(r + 1) * H] = out_t.T


def kernel(x, W_enc, b_enc, ln_g, ln_b, Wq, Wk, Wv, bq, bk, bv, Wo, bo):
    grid = (B, T // TB)
    full = lambda b, t: (0, 0)
    gf, rf = pl.pallas_call(
        _body,
        grid=grid,
        in_specs=[
            pl.BlockSpec((1, TB, R * Cg), lambda b, t: (b, t, 0)),
            pl.BlockSpec((R, Cg, H), lambda b, t: (0, 0, 0)),
            pl.BlockSpec((R, H), full),
            pl.BlockSpec((R, H), full),
            pl.BlockSpec((R, H), full),
            pl.BlockSpec((H, H), full),
            pl.BlockSpec((H, H), full),
            pl.BlockSpec((H, H), full),
            pl.BlockSpec((H, 1), full),
            pl.BlockSpec((H, 1), full),
            pl.BlockSpec((H, 1), full),
            pl.BlockSpec((H, H), full),
            pl.BlockSpec((H, 1), full),
        ],
        out_specs=[
            pl.BlockSpec((1, TB, R * H), lambda b, t: (b, t, 0)),
            pl.BlockSpec((1, TB, R, H), lambda b, t: (b, t, 0, 0)),
        ],
        out_shape=[
            jax.ShapeDtypeStruct((B, T, R * H), jnp.float32),
            jax.ShapeDtypeStruct((B, T, R, H), jnp.float32),
        ],
    )(x, W_enc, b_enc, ln_g, ln_b, Wq, Wk, Wv,
      bq.reshape(H, 1), bk.reshape(H, 1), bv.reshape(H, 1),
      Wo, bo.reshape(H, 1))
    return gf, rf
